# Initial kernel scaffold; baseline (speedup 1.0000x reference)
import jax
import jax.numpy as jnp
from jax.experimental import pallas as pl


def kernel(node_inputs, adj, hints, lengths, W_enc_in, W_enc_hint, W_edge, W_m1, W_m2, W_o1, W_o2, W_dec_out, W_dec_hint):
    raise NotImplementedError("write your pallas kernel here")


# speedup vs baseline: 1.0787x; 1.0787x over previous
import jax
import jax.numpy as jnp
from jax.experimental import pallas as pl
from jax.experimental.pallas import tpu as pltpu

_NEG = -1e9
_SRC_BLK = 8


def _net_kernel(sstar_ref, x_ref, adj_ref, adjt_ref, hints_ref, vecs_ref,
                wm1_ref, wm2_ref, wo1_ref, wo2_ref, out_ref):
    b = pl.program_id(0)
    n = adj_ref.shape[1]
    t_total = hints_ref.shape[2]
    h = wo2_ref.shape[1]
    nsteps = max(1, t_total - 1)

    x = x_ref[0]
    adjb = adj_ref[0]
    hb = hints_ref[0]
    vecs = vecs_ref[:, :]
    v1h = vecs[1:2]
    v2h = vecs[3:4]
    voh = vecs[5:6]
    we = vecs[6:7]
    wdec = vecs[7:8]
    wm1 = wm1_ref[:, :]
    wm2 = wm2_ref[:, :]
    wo1 = wo1_ref[:, :]
    wo2 = wo2_ref[:, :]

    penalty = jnp.where(adjb > 0.5, 0.0, _NEG)
    empty = jnp.max(adjt_ref[0], axis=1, keepdims=True) <= 0.5
    sstar = sstar_ref[b]

    xm1 = x * vecs[0:1]
    xm2 = x * vecs[2:3]
    xo = x * vecs[4:5]

    def step(i, hidden):
        onehot = (jax.lax.broadcasted_iota(jnp.int32, (t_total, 1), 0)
                  == i).astype(jnp.float32)
        hc = jnp.dot(hb, onehot, preferred_element_type=jnp.float32)
        m1 = xm1 + hc * v1h + jnp.dot(hidden, wm1,
                                      preferred_element_type=jnp.float32)
        m2 = xm2 + hc * v2h + jnp.dot(hidden, wm2,
                                      preferred_element_type=jnp.float32)
        zo = xo + hc * voh + jnp.dot(hidden, wo1,
                                     preferred_element_type=jnp.float32)

        m = jnp.full((n, h), _NEG, dtype=jnp.float32)
        for s0 in range(0, n, _SRC_BLK):
            a = adjb[s0:s0 + _SRC_BLK]
            p = penalty[s0:s0 + _SRC_BLK]
            m1b = m1[s0:s0 + _SRC_BLK]
            tblk = a[:, :, None] * we[0] + (p[:, :, None] + m1b[:, None, :])
            m = jnp.maximum(m, jnp.max(tblk, axis=0))
        msgs = jnp.where(empty, _NEG, jax.nn.relu(m2 + m))
        h_new = jax.nn.relu(zo + jnp.dot(msgs, wo2,
                                         preferred_element_type=jnp.float32))
        out_cand = jnp.sum(h_new * wdec, axis=1)

        @pl.when(i == sstar)
        def _():
            out_ref[0, 0, :] = out_cand

        return h_new

    jax.lax.fori_loop(0, nsteps, step, jnp.zeros((n, h), jnp.float32))


def kernel(node_inputs, adj, hints, lengths, W_enc_in, W_enc_hint, W_edge,
           W_m1, W_m2, W_o1, W_o2, W_dec_out, W_dec_hint):
    B, N, _ = node_inputs.shape
    T = hints.shape[0]
    H = W_o1.shape[1]

    hints_nt = jnp.transpose(hints, (1, 2, 0))
    adj_t = jnp.swapaxes(adj, 1, 2)
    vecs = jnp.concatenate([
        W_enc_in @ W_m1[:H], W_enc_hint @ W_m1[:H],
        W_enc_in @ W_m2[:H], W_enc_hint @ W_m2[:H],
        W_enc_in @ W_o1[:H], W_enc_hint @ W_o1[:H],
        W_edge, W_dec_out.T,
    ], axis=0)
    sstar = jnp.clip(lengths - 2, 0, max(0, T - 2)).astype(jnp.int32)

    return pl.pallas_call(
        _net_kernel,
        grid=(B,),
        in_specs=[
            pl.BlockSpec(memory_space=pltpu.SMEM),
            pl.BlockSpec((1, N, 1), lambda b: (b, 0, 0)),
            pl.BlockSpec((1, N, N), lambda b: (b, 0, 0)),
            pl.BlockSpec((1, N, N), lambda b: (b, 0, 0)),
            pl.BlockSpec((1, N, T), lambda b: (b, 0, 0)),
            pl.BlockSpec((8, H), lambda b: (0, 0)),
            pl.BlockSpec((H, H), lambda b: (0, 0)),
            pl.BlockSpec((H, H), lambda b: (0, 0)),
            pl.BlockSpec((H, H), lambda b: (0, 0)),
            pl.BlockSpec((H, H), lambda b: (0, 0)),
        ],
        out_specs=pl.BlockSpec((1, 1, N), lambda b: (b, 0, 0)),
        out_shape=jax.ShapeDtypeStruct((B, 1, N), jnp.float32),
        compiler_params=pltpu.CompilerParams(
            dimension_semantics=("parallel",)),
    )(sstar, node_inputs, adj, adj_t, hints_nt, vecs,
      W_m1[H:], W_m2[H:], W_o1[H:], W_o2)[:, 0, :]


# speedup vs baseline: 2.4918x; 2.3100x over previous
import jax
import jax.numpy as jnp
from jax.experimental import pallas as pl
from jax.experimental.pallas import tpu as pltpu

_NEG = -1e9
_SRC_BLK = 8


def _net_kernel(sstar_ref, x_ref, adj_ref, adjt_ref, hints_ref, vecs_ref,
                wm1_ref, wm2_ref, wo1_ref, wo2_ref, out_ref, awe_ref):
    b = pl.program_id(0)
    n = adj_ref.shape[1]
    t_total = hints_ref.shape[2]
    h = wo2_ref.shape[1]
    nsteps = max(1, t_total - 1)

    x = x_ref[0]
    adjb = adj_ref[0]
    hb = hints_ref[0]
    vecs = vecs_ref[:, :]
    v1h = vecs[1:2]
    v2h = vecs[3:4]
    voh = vecs[5:6]
    we = vecs[6:7]
    wdec = vecs[7:8]
    wm1 = wm1_ref[:, :]
    wm2 = wm2_ref[:, :]
    wo1 = wo1_ref[:, :]
    wo2 = wo2_ref[:, :]

    penalty = jnp.where(adjb > 0.5, 0.0, _NEG)
    empty = jnp.max(adjt_ref[0], axis=1, keepdims=True) <= 0.5
    sstar = sstar_ref[b]

    xm1 = x * vecs[0:1]
    xm2 = x * vecs[2:3]
    xo = x * vecs[4:5]

    for s0 in range(0, n, _SRC_BLK):
        a = adjb[s0:s0 + _SRC_BLK]
        p = penalty[s0:s0 + _SRC_BLK]
        awe_ref[s0:s0 + _SRC_BLK] = (a[:, :, None] * we[0] + p[:, :, None])

    def step(i, hidden):
        onehot = (jax.lax.broadcasted_iota(jnp.int32, (t_total, 1), 0)
                  == i).astype(jnp.float32)
        hc = jnp.dot(hb, onehot, preferred_element_type=jnp.float32)
        m1 = xm1 + hc * v1h + jnp.dot(hidden, wm1,
                                      preferred_element_type=jnp.float32)
        m2 = xm2 + hc * v2h + jnp.dot(hidden, wm2,
                                      preferred_element_type=jnp.float32)
        zo = xo + hc * voh + jnp.dot(hidden, wo1,
                                     preferred_element_type=jnp.float32)

        m = jnp.full((n, h), _NEG, dtype=jnp.float32)
        for s0 in range(0, n, _SRC_BLK):
            m1b = m1[s0:s0 + _SRC_BLK]
            tblk = awe_ref[s0:s0 + _SRC_BLK] + m1b[:, None, :]
            m = jnp.maximum(m, jnp.max(tblk, axis=0))
        msgs = jnp.where(empty, _NEG, jax.nn.relu(m2 + m))
        h_new = jax.nn.relu(zo + jnp.dot(msgs, wo2,
                                         preferred_element_type=jnp.float32))
        out_cand = jnp.sum(h_new * wdec, axis=1)

        @pl.when(i == sstar)
        def _():
            out_ref[0, 0, :] = out_cand

        return h_new

    jax.lax.fori_loop(0, nsteps, step, jnp.zeros((n, h), jnp.float32))


def kernel(node_inputs, adj, hints, lengths, W_enc_in, W_enc_hint, W_edge,
           W_m1, W_m2, W_o1, W_o2, W_dec_out, W_dec_hint):
    B, N, _ = node_inputs.shape
    T = hints.shape[0]
    H = W_o1.shape[1]

    hints_nt = jnp.transpose(hints, (1, 2, 0))
    adj_t = jnp.swapaxes(adj, 1, 2)
    vecs = jnp.concatenate([
        W_enc_in @ W_m1[:H], W_enc_hint @ W_m1[:H],
        W_enc_in @ W_m2[:H], W_enc_hint @ W_m2[:H],
        W_enc_in @ W_o1[:H], W_enc_hint @ W_o1[:H],
        W_edge, W_dec_out.T,
    ], axis=0)
    sstar = jnp.clip(lengths - 2, 0, max(0, T - 2)).astype(jnp.int32)

    return pl.pallas_call(
        _net_kernel,
        grid=(B,),
        in_specs=[
            pl.BlockSpec(memory_space=pltpu.SMEM),
            pl.BlockSpec((1, N, 1), lambda b: (b, 0, 0)),
            pl.BlockSpec((1, N, N), lambda b: (b, 0, 0)),
            pl.BlockSpec((1, N, N), lambda b: (b, 0, 0)),
            pl.BlockSpec((1, N, T), lambda b: (b, 0, 0)),
            pl.BlockSpec((8, H), lambda b: (0, 0)),
            pl.BlockSpec((H, H), lambda b: (0, 0)),
            pl.BlockSpec((H, H), lambda b: (0, 0)),
            pl.BlockSpec((H, H), lambda b: (0, 0)),
            pl.BlockSpec((H, H), lambda b: (0, 0)),
        ],
        out_specs=pl.BlockSpec((1, 1, N), lambda b: (b, 0, 0)),
        out_shape=jax.ShapeDtypeStruct((B, 1, N), jnp.float32),
        scratch_shapes=[pltpu.VMEM((N, N, H), jnp.float32)],
        compiler_params=pltpu.CompilerParams(
            dimension_semantics=("parallel",)),
    )(sstar, node_inputs, adj, adj_t, hints_nt, vecs,
      W_m1[H:], W_m2[H:], W_o1[H:], W_o2)[:, 0, :]


# speedup vs baseline: 3.5929x; 1.4419x over previous
import jax
import jax.numpy as jnp
from jax.experimental import pallas as pl
from jax.experimental.pallas import tpu as pltpu

_NEG = -1e9
_SRC_BLK = 8


def _net_kernel(sstar_ref, x_ref, adj_ref, adjt_ref, hints_ref, vecs_ref,
                wcat_ref, wo2_ref, out_ref, awe_ref):
    b = pl.program_id(0)
    n = adj_ref.shape[1]
    t_total = hints_ref.shape[2]
    h = wo2_ref.shape[1]
    nsteps = max(1, t_total - 1)

    x = x_ref[0]
    adjb = adj_ref[0]
    hb = hints_ref[0]
    vecs = vecs_ref[:, :]
    v1h = vecs[1:2]
    v2h = vecs[3:4]
    voh = vecs[5:6]
    we = vecs[6:7]
    wdec = vecs[7:8]
    wcat = wcat_ref[:, :]
    wo2 = wo2_ref[:, :]

    empty = jnp.max(adjt_ref[0], axis=1, keepdims=True) <= 0.5
    sstar = sstar_ref[b]

    xm1 = x * vecs[0:1]
    xm2 = x * vecs[2:3]
    xo = x * vecs[4:5]

    adjtb = adjt_ref[0]
    pt = jnp.where(adjtb > 0.5, 0.0, _NEG)
    for j0 in range(0, n, _SRC_BLK):
        a = adjtb[j0:j0 + _SRC_BLK]
        p = pt[j0:j0 + _SRC_BLK]
        awe_ref[j0:j0 + _SRC_BLK] = (a[:, :, None] * we[0] + p[:, :, None])

    def step(i, hidden):
        onehot = (jax.lax.broadcasted_iota(jnp.int32, (t_total, 1), 0)
                  == i).astype(jnp.float32)
        hc = jnp.dot(hb, onehot, preferred_element_type=jnp.float32)
        mm = jnp.dot(hidden, wcat, preferred_element_type=jnp.float32)
        m1 = xm1 + hc * v1h + mm[:, :h]
        m2 = xm2 + hc * v2h + mm[:, h:2 * h]
        zo = xo + hc * voh + mm[:, 2 * h:]

        parts = []
        for j0 in range(0, n, _SRC_BLK):
            tblk = awe_ref[j0:j0 + _SRC_BLK] + m1[None, :, :]
            parts.append(jnp.max(tblk, axis=1))
        m = jnp.concatenate(parts, axis=0)
        msgs = jnp.where(empty, _NEG, jax.nn.relu(m2 + m))
        h_new = jax.nn.relu(zo + jnp.dot(msgs, wo2,
                                         preferred_element_type=jnp.float32))
        out_cand = jnp.sum(h_new * wdec, axis=1)

        @pl.when(i == sstar)
        def _():
            out_ref[0, 0, :] = out_cand

        return h_new

    jax.lax.fori_loop(0, nsteps, step, jnp.zeros((n, h), jnp.float32))


def kernel(node_inputs, adj, hints, lengths, W_enc_in, W_enc_hint, W_edge,
           W_m1, W_m2, W_o1, W_o2, W_dec_out, W_dec_hint):
    B, N, _ = node_inputs.shape
    T = hints.shape[0]
    H = W_o1.shape[1]

    hints_nt = jnp.transpose(hints, (1, 2, 0))
    adj_t = jnp.swapaxes(adj, 1, 2)
    vecs = jnp.concatenate([
        W_enc_in @ W_m1[:H], W_enc_hint @ W_m1[:H],
        W_enc_in @ W_m2[:H], W_enc_hint @ W_m2[:H],
        W_enc_in @ W_o1[:H], W_enc_hint @ W_o1[:H],
        W_edge, W_dec_out.T,
    ], axis=0)
    sstar = jnp.clip(lengths - 2, 0, max(0, T - 2)).astype(jnp.int32)

    return pl.pallas_call(
        _net_kernel,
        grid=(B,),
        in_specs=[
            pl.BlockSpec(memory_space=pltpu.SMEM),
            pl.BlockSpec((1, N, 1), lambda b: (b, 0, 0)),
            pl.BlockSpec((1, N, N), lambda b: (b, 0, 0)),
            pl.BlockSpec((1, N, N), lambda b: (b, 0, 0)),
            pl.BlockSpec((1, N, T), lambda b: (b, 0, 0)),
            pl.BlockSpec((8, H), lambda b: (0, 0)),
            pl.BlockSpec((H, 3 * H), lambda b: (0, 0)),
            pl.BlockSpec((H, H), lambda b: (0, 0)),
        ],
        out_specs=pl.BlockSpec((1, 1, N), lambda b: (b, 0, 0)),
        out_shape=jax.ShapeDtypeStruct((B, 1, N), jnp.float32),
        scratch_shapes=[pltpu.VMEM((N, N, H), jnp.float32)],
        compiler_params=pltpu.CompilerParams(
            dimension_semantics=("parallel",)),
    )(sstar, node_inputs, adj, adj_t, hints_nt, vecs,
      jnp.concatenate([W_m1[H:], W_m2[H:], W_o1[H:]], axis=1), W_o2)[:, 0, :]
